# Initial kernel scaffold; baseline (speedup 1.0000x reference)
import jax
import jax.numpy as jnp
from jax.experimental import pallas as pl


def kernel(x, centers):
    raise NotImplementedError("write your pallas kernel here")


# speedup vs baseline: 45.6916x; 45.6916x over previous
import functools

import jax
import jax.numpy as jnp
from jax import lax
from jax.experimental import pallas as pl
from jax.experimental.pallas import tpu as pltpu
from jax.experimental.pallas import tpu_sc as plsc

_LANES = 16


def _sc_quantize(total, n_workers):
    chunk = total // n_workers
    n_vecs = chunk // _LANES
    mesh = plsc.VectorSubcoreMesh(core_axis_name="c", subcore_axis_name="s")

    @functools.partial(
        pl.kernel,
        out_type=[
            jax.ShapeDtypeStruct((total,), jnp.float32),
            jax.ShapeDtypeStruct((total,), jnp.float32),
            jax.ShapeDtypeStruct((total,), jnp.int32),
        ],
        mesh=mesh,
        compiler_params=pltpu.CompilerParams(needs_layout_passes=False),
        scratch_types=[
            pltpu.VMEM((chunk,), jnp.float32),
            pltpu.VMEM((64,), jnp.float32),
            pltpu.VMEM((64,), jnp.float32),
            pltpu.VMEM((chunk,), jnp.float32),
            pltpu.VMEM((chunk,), jnp.int32),
        ],
    )
    def body(x_hbm, mids_hbm, cent_hbm, ste_hbm, hard_hbm, idx_hbm,
             x_v, mids_v, cent_v, hard_v, idx_v):
        wid = lax.axis_index("s") * 2 + lax.axis_index("c")
        base = wid * chunk
        pltpu.sync_copy(mids_hbm, mids_v)
        pltpu.sync_copy(cent_hbm, cent_v)
        pltpu.sync_copy(x_hbm.at[pl.ds(base, chunk)], x_v)

        def step(i, _):
            off = i * _LANES
            xv = x_v[pl.ds(off, _LANES)]
            pos = jnp.zeros((_LANES,), jnp.int32)
            for s in (32, 16, 8, 4, 2, 1):
                t = plsc.load_gather(mids_v, [pos + (s - 1)])
                pos = pos + jnp.where(xv > t, jnp.int32(s), jnp.int32(0))
            hard = plsc.load_gather(cent_v, [pos])
            hard_v[pl.ds(off, _LANES)] = hard
            idx_v[pl.ds(off, _LANES)] = pos
            return _

        lax.fori_loop(0, n_vecs, step, None)

        pltpu.sync_copy(hard_v, ste_hbm.at[pl.ds(base, chunk)])
        pltpu.sync_copy(hard_v, hard_hbm.at[pl.ds(base, chunk)])
        pltpu.sync_copy(idx_v, idx_hbm.at[pl.ds(base, chunk)])

    return body


def kernel(x, centers):
    n, c, h, w = x.shape
    total = n * c * h * w
    xf = x.reshape(total)
    mids = jnp.concatenate(
        [(centers[:-1] + centers[1:]) * 0.5, jnp.full((1,), jnp.inf, jnp.float32)]
    )
    n_workers = 32
    ste, hard, idx = _sc_quantize(total, n_workers)(xf, mids, centers)
    shape = (n, c, h, w)
    return (ste.reshape(shape), hard.reshape(shape), idx.reshape(shape))


# speedup vs baseline: 67.8290x; 1.4845x over previous
import functools

import jax
import jax.numpy as jnp
from jax import lax
from jax.experimental import pallas as pl
from jax.experimental.pallas import tpu as pltpu
from jax.experimental.pallas import tpu_sc as plsc

_LANES = 16


def _sc_quantize(total, n_workers):
    chunk = total // n_workers
    n_vecs = chunk // _LANES
    mesh = plsc.VectorSubcoreMesh(core_axis_name="c", subcore_axis_name="s")

    @functools.partial(
        pl.kernel,
        out_type=[
            jax.ShapeDtypeStruct((total,), jnp.float32),
            jax.ShapeDtypeStruct((total,), jnp.float32),
            jax.ShapeDtypeStruct((total,), jnp.int32),
        ],
        mesh=mesh,
        compiler_params=pltpu.CompilerParams(needs_layout_passes=False),
        scratch_types=[
            pltpu.VMEM((chunk,), jnp.float32),
            pltpu.VMEM((64,), jnp.float32),
            pltpu.VMEM((64,), jnp.float32),
            pltpu.VMEM((chunk,), jnp.float32),
            pltpu.VMEM((chunk,), jnp.int32),
        ],
    )
    def body(x_hbm, mids_hbm, cent_hbm, ste_hbm, hard_hbm, idx_hbm,
             x_v, mids_v, cent_v, hard_v, idx_v):
        wid = lax.axis_index("s") * 2 + lax.axis_index("c")
        base = wid * chunk
        pltpu.sync_copy(mids_hbm, mids_v)
        pltpu.sync_copy(cent_hbm, cent_v)
        pltpu.sync_copy(x_hbm.at[pl.ds(base, chunk)], x_v)

        @plsc.parallel_loop(0, n_vecs, 1, unroll=8)
        def step(i):
            off = i * _LANES
            xv = x_v[pl.ds(off, _LANES)]
            pos = jnp.zeros((_LANES,), jnp.int32)
            for s in (32, 16, 8, 4, 2, 1):
                t = plsc.load_gather(mids_v, [pos + (s - 1)])
                pos = pos + jnp.where(xv > t, jnp.int32(s), jnp.int32(0))
            hard = plsc.load_gather(cent_v, [pos])
            hard_v[pl.ds(off, _LANES)] = hard
            idx_v[pl.ds(off, _LANES)] = pos

        pltpu.sync_copy(hard_v, ste_hbm.at[pl.ds(base, chunk)])
        pltpu.sync_copy(hard_v, hard_hbm.at[pl.ds(base, chunk)])
        pltpu.sync_copy(idx_v, idx_hbm.at[pl.ds(base, chunk)])

    return body


def kernel(x, centers):
    n, c, h, w = x.shape
    total = n * c * h * w
    xf = x.reshape(total)
    mids = jnp.concatenate(
        [(centers[:-1] + centers[1:]) * 0.5, jnp.full((1,), jnp.inf, jnp.float32)]
    )
    n_workers = 32
    ste, hard, idx = _sc_quantize(total, n_workers)(xf, mids, centers)
    shape = (n, c, h, w)
    return (ste.reshape(shape), hard.reshape(shape), idx.reshape(shape))
